# baseline (speedup 1.0000x reference)
import functools

import jax
import jax.numpy as jnp
from jax import lax
from jax.experimental import pallas as pl
from jax.experimental.pallas import tpu as pltpu
from jax.experimental.pallas import tpu_sc as plsc

D = 64
DP = 128


@functools.lru_cache(maxsize=None)
def _build_transpose(V: int):
    info = plsc.get_sparse_core_info()
    NC, NS = info.num_cores, info.num_subcores
    NW = NC * NS
    W = 256
    NB_FULL = V // W
    V_TAIL = (V // DP - 1) * DP
    N_I = (NB_FULL + NW - 1) // NW

    mesh = plsc.VectorSubcoreMesh(core_axis_name="c", subcore_axis_name="s")

    @functools.partial(
        pl.kernel,
        mesh=mesh,
        compiler_params=pltpu.CompilerParams(
            use_tc_tiling_on_sc=True, needs_layout_passes=False
        ),
        out_type=jax.ShapeDtypeStruct((V, DP), jnp.float32),
        scratch_types=[
            pltpu.VMEM((2, D, W + 1), jnp.float32),
            pltpu.VMEM((2, DP, DP), jnp.float32),
            pltpu.VMEM((D, 192), jnp.float32),
            pltpu.SemaphoreType.DMA,
            pltpu.SemaphoreType.DMA,
            pltpu.SemaphoreType.DMA,
            pltpu.SemaphoreType.DMA,
        ],
    )
    def transpose_kernel(tt_hbm, out_hbm, src_v, dst_v, tsrc_v,
                         rsem0, rsem1, wsem0, wsem1):
        wid = lax.axis_index("s") * NC + lax.axis_index("c")
        IOTA = lax.iota(jnp.int32, 16)
        rsems = (rsem0, rsem1)
        wsems = (wsem0, wsem1)

        def issue_reads(base, par):
            for dh in range(8):
                pltpu.async_copy(
                    tt_hbm.at[pl.ds(dh * 8, 8), pl.ds(base, W)],
                    src_v.at[par, pl.ds(dh * 8, 8), pl.ds(0, W)],
                    rsems[par],
                )

        def transpose(sv, dv, c0, width):
            iotas = [IOTA + d0 for d0 in range(0, D, 16)]

            def rbody(r4, carry):
                for u in range(4):
                    r = r4 * 4 + u
                    cidx = jnp.full((16,), c0, jnp.int32) + r
                    for i, d0 in enumerate(range(0, D, 16)):
                        x = plsc.load_gather(sv, [iotas[i], cidx])
                        dv[r, pl.ds(d0, 16)] = x
                return carry

            lax.fori_loop(0, width // 4, rbody, jnp.int32(0))

        def body(i2, carry):
            for par in range(2):
                j = 2 * i2 + par
                b = wid + j * NW

                @pl.when(b < NB_FULL)
                def _():
                    bn = b + NW

                    @pl.when(bn < NB_FULL)
                    def _():
                        issue_reads(bn * W, (par + 1) % 2)

                    pltpu.make_async_copy(
                        tt_hbm.at[pl.ds(0, D), pl.ds(0, W)],
                        src_v.at[par, :, pl.ds(0, W)], rsems[par],
                    ).wait()

                    for h in range(2):
                        @pl.when(j >= 1)
                        def _():
                            pltpu.make_async_copy(
                                out_hbm.at[pl.ds(0, DP)],
                                dst_v.at[h, :, pl.ds(0, DP)], wsems[h],
                            ).wait()

                        transpose(src_v.at[par], dst_v.at[h], h * DP, DP)
                        pltpu.async_copy(
                            dst_v.at[h, :, pl.ds(0, DP)],
                            out_hbm.at[pl.ds(b * W + h * DP, DP)],
                            wsems[h],
                        )

            return carry

        issue_reads(wid * W, 0)
        lax.fori_loop(0, (N_I + 1) // 2, body, jnp.int32(0))

        n_b = (NB_FULL - wid + NW - 1) // NW

        @pl.when(n_b >= 1)
        def _():
            pltpu.make_async_copy(
                out_hbm.at[pl.ds(0, DP)], dst_v.at[0, :, pl.ds(0, DP)], wsem0
            ).wait()
            pltpu.make_async_copy(
                out_hbm.at[pl.ds(0, DP)], dst_v.at[1, :, pl.ds(0, DP)], wsem1
            ).wait()

        @pl.when(wid == NB_FULL % NW)
        def _():
            waits = []
            for dh in range(8):
                waits.append(pltpu.async_copy(
                    tt_hbm.at[pl.ds(dh * 8, 8), pl.ds(V_TAIL, V - V_TAIL)],
                    tsrc_v.at[pl.ds(dh * 8, 8), :],
                    rsem0,
                ))
            for h in waits:
                h.wait()
            transpose(tsrc_v, dst_v.at[0], 0, DP)
            pltpu.sync_copy(
                dst_v.at[0, :, pl.ds(0, DP)], out_hbm.at[pl.ds(V_TAIL, DP)]
            )
            transpose(tsrc_v, dst_v.at[1], DP, V - V_TAIL - DP)
            pltpu.sync_copy(
                dst_v.at[1, pl.ds(0, V - V_TAIL - DP), pl.ds(0, DP)],
                out_hbm.at[pl.ds(V_TAIL + DP, V - V_TAIL - DP)],
            )

    return transpose_kernel


@functools.lru_cache(maxsize=None)
def _build_gather(T: int, V: int):
    info = plsc.get_sparse_core_info()
    NC, NS = info.num_cores, info.num_subcores
    NW = NC * NS
    assert T % NW == 0
    b_per_w = T // NW
    C = 400
    assert b_per_w % C == 0
    n_chunks = b_per_w // C

    mesh = plsc.VectorSubcoreMesh(core_axis_name="c", subcore_axis_name="s")

    @functools.partial(
        pl.kernel,
        mesh=mesh,
        compiler_params=pltpu.CompilerParams(use_tc_tiling_on_sc=True),
        out_type=jax.ShapeDtypeStruct((T, DP), jnp.float32),
        scratch_types=[
            pltpu.VMEM((b_per_w,), jnp.int32),
            pltpu.VMEM((2, C, DP), jnp.float32),
            pltpu.SemaphoreType.DMA,
            pltpu.SemaphoreType.DMA,
        ],
    )
    def gather_kernel(table_hbm, idx_hbm, out_hbm, idx_v, rows_v, gsem, ssem):
        wid = lax.axis_index("s") * NC + lax.axis_index("c")
        base = wid * b_per_w
        pltpu.sync_copy(idx_hbm.at[pl.ds(base, b_per_w)], idx_v)
        gathers = [None] * n_chunks
        scatters = [None] * n_chunks
        gathers[0] = pltpu.async_copy(
            table_hbm.at[idx_v.at[pl.ds(0, C)]], rows_v.at[0], gsem
        )
        for j in range(n_chunks):
            if j + 1 < n_chunks:
                if j >= 1:
                    scatters[j - 1].wait()
                gathers[j + 1] = pltpu.async_copy(
                    table_hbm.at[idx_v.at[pl.ds((j + 1) * C, C)]],
                    rows_v.at[(j + 1) % 2],
                    gsem,
                )
            gathers[j].wait()
            scatters[j] = pltpu.async_copy(
                rows_v.at[j % 2], out_hbm.at[pl.ds(base + j * C, C)], ssem
            )
        scatters[n_chunks - 2].wait()
        scatters[n_chunks - 1].wait()

    return gather_kernel


def kernel(caption, table):
    B, L = caption.shape
    T = B * L
    V = table.shape[0]
    idx = jnp.swapaxes(caption, 0, 1).reshape(T).astype(jnp.int32)
    tt = jnp.swapaxes(table, 0, 1)
    table_rm = _build_transpose(V)(tt)
    out = _build_gather(T, V)(table_rm, idx)
    out64 = out[:, :D]
    return jnp.swapaxes(out64.reshape(L, B, D), 0, 1)


# speedup vs baseline: 2.4755x; 2.4755x over previous
import functools

import jax
import jax.numpy as jnp
from jax import lax
from jax.experimental import pallas as pl
from jax.experimental.pallas import tpu as pltpu
from jax.experimental.pallas import tpu_sc as plsc

D = 64
DP = 128


@functools.lru_cache(maxsize=None)
def _build(T: int, V: int):
    info = plsc.get_sparse_core_info()
    NC, NS = info.num_cores, info.num_subcores
    NW = NC * NS
    assert T % NW == 0
    b_per_w = T // NW
    C = 400
    assert b_per_w % C == 0
    n_chunks = b_per_w // C

    mesh = plsc.VectorSubcoreMesh(core_axis_name="c", subcore_axis_name="s")

    @functools.partial(
        pl.kernel,
        mesh=mesh,
        compiler_params=pltpu.CompilerParams(use_tc_tiling_on_sc=True),
        out_type=jax.ShapeDtypeStruct((T, DP), jnp.float32),
        scratch_types=[
            pltpu.VMEM((b_per_w,), jnp.int32),
            pltpu.VMEM((2, C, DP), jnp.float32),
            pltpu.SemaphoreType.DMA,
            pltpu.SemaphoreType.DMA,
        ],
    )
    def gather_kernel(table_hbm, idx_hbm, out_hbm, idx_v, rows_v, gsem, ssem):
        wid = lax.axis_index("s") * NC + lax.axis_index("c")
        base = wid * b_per_w
        pltpu.sync_copy(idx_hbm.at[pl.ds(base, b_per_w)], idx_v)
        gathers = [None] * n_chunks
        scatters = [None] * n_chunks
        gathers[0] = pltpu.async_copy(
            table_hbm.at[idx_v.at[pl.ds(0, C)]], rows_v.at[0], gsem
        )
        for j in range(n_chunks):
            if j + 1 < n_chunks:
                if j >= 1:
                    scatters[j - 1].wait()
                gathers[j + 1] = pltpu.async_copy(
                    table_hbm.at[idx_v.at[pl.ds((j + 1) * C, C)]],
                    rows_v.at[(j + 1) % 2],
                    gsem,
                )
            gathers[j].wait()
            scatters[j] = pltpu.async_copy(
                rows_v.at[j % 2], out_hbm.at[pl.ds(base + j * C, C)], ssem
            )
        scatters[n_chunks - 2].wait()
        scatters[n_chunks - 1].wait()

    return gather_kernel


def kernel(caption, table):
    B, L = caption.shape
    T = B * L
    idx = jnp.swapaxes(caption, 0, 1).reshape(T).astype(jnp.int32)
    table_p = jnp.pad(table, ((0, 0), (0, DP - D)))
    out = _build(T, table.shape[0])(table_p, idx)
    out64 = out[:, :D]
    return jnp.swapaxes(out64.reshape(L, B, D), 0, 1)
